# Initial kernel scaffold; baseline (speedup 1.0000x reference)
import jax
import jax.numpy as jnp
from jax.experimental import pallas as pl


def kernel(positive_triplets, negative_triplets, entity_emb, relation_emb):
    raise NotImplementedError("write your pallas kernel here")


# speedup vs baseline: 1.8417x; 1.8417x over previous
import functools

import jax
import jax.numpy as jnp
from jax import lax
from jax.experimental import pallas as pl
from jax.experimental.pallas import tpu as pltpu
from jax.experimental.pallas import tpu_sc as plsc

ENTITY_SIZE = 1000000
EMB = 64
BATCH = 16384
TOTAL = 2 * BATCH

NC = 2
NS = 16
NW = NC * NS
L = 16

PER_W = TOTAL // NW
CH = 256
GROUPS = CH // L


def _rsqrt_nr(x):
    i = plsc.bitcast(x, jnp.int32)
    i = jnp.int32(0x5F3759DF) - lax.shift_right_logical(i, 1)
    y = plsc.bitcast(i, jnp.float32)
    half = x * jnp.float32(0.5)
    for _ in range(3):
        y = y * (jnp.float32(1.5) - half * y * y)
    return y


def _body(hidx_hbm, ridx_hbm, tidx_hbm, ent_hbm, rel_hbm, out_hbm,
          hidx_v, ridx_v, tidx_v, hrows, rrows, trows, out_v,
          semh, semr, semt):
    wid = lax.axis_index("s") * NC + lax.axis_index("c")
    lane = lax.broadcasted_iota(jnp.int32, (L,), 0)
    zeros = jnp.zeros((L,), jnp.float32)
    ones = jnp.full((L,), 1.0, jnp.float32)
    last = jnp.full((L,), ENTITY_SIZE - 1, jnp.int32)

    for chunk in range(PER_W // CH):
        base = wid * PER_W + chunk * CH
        pltpu.sync_copy(hidx_hbm.at[pl.ds(base, CH)], hidx_v)
        pltpu.sync_copy(ridx_hbm.at[pl.ds(base, CH)], ridx_v)
        pltpu.sync_copy(tidx_hbm.at[pl.ds(base, CH)], tidx_v)
        ch = pltpu.async_copy(ent_hbm.at[hidx_v], hrows, semh)
        cr = pltpu.async_copy(rel_hbm.at[ridx_v], rrows, semr)
        ct = pltpu.async_copy(ent_hbm.at[tidx_v], trows, semt)
        ch.wait()
        cr.wait()
        ct.wait()

        def group(g, carry):
            rows = g * L + lane
            hidx = hidx_v[pl.ds(g * L, L)]
            tidx = tidx_v[pl.ds(g * L, L)]

            def norms(c, accs):
                ah, at = accs
                cols = jnp.full((L,), 0, jnp.int32) + c
                gh = plsc.load_gather(hrows, [rows, cols])
                gt = plsc.load_gather(trows, [rows, cols])
                return ah + gh * gh, at + gt * gt

            ah, at = lax.fori_loop(0, EMB, norms, (zeros, zeros))
            sh = jnp.where(hidx == last, ones, _rsqrt_nr(ah))
            st = jnp.where(tidx == last, ones, _rsqrt_nr(at))

            def dist2(c, acc):
                cols = jnp.full((L,), 0, jnp.int32) + c
                gh = plsc.load_gather(hrows, [rows, cols])
                gr = plsc.load_gather(rrows, [rows, cols])
                gt = plsc.load_gather(trows, [rows, cols])
                d = gh * sh + gr - gt * st
                return acc + d * d

            acc = lax.fori_loop(0, EMB, dist2, zeros)
            dist = jnp.where(acc > 0, acc * _rsqrt_nr(acc), zeros)
            out_v[pl.ds(g * L, L)] = dist
            return carry

        lax.fori_loop(0, GROUPS, group, 0)
        pltpu.sync_copy(out_v, out_hbm.at[pl.ds(base, CH)])


@functools.partial(jax.jit, donate_argnums=())
def _trans_e(hidx, ridx, tidx, ent, rel):
    run = functools.partial(
        pl.kernel,
        out_type=jax.ShapeDtypeStruct((TOTAL,), jnp.float32),
        mesh=plsc.VectorSubcoreMesh(core_axis_name="c", subcore_axis_name="s"),
        scratch_types=[
            pltpu.VMEM((CH,), jnp.int32),
            pltpu.VMEM((CH,), jnp.int32),
            pltpu.VMEM((CH,), jnp.int32),
            pltpu.VMEM((CH, EMB), jnp.float32),
            pltpu.VMEM((CH, EMB), jnp.float32),
            pltpu.VMEM((CH, EMB), jnp.float32),
            pltpu.VMEM((CH,), jnp.float32),
            pltpu.SemaphoreType.DMA,
            pltpu.SemaphoreType.DMA,
            pltpu.SemaphoreType.DMA,
        ],
        compiler_params=pltpu.CompilerParams(
            needs_layout_passes=False, use_tc_tiling_on_sc=False
        ),
    )(_body)
    return run(hidx, ridx, tidx, ent, rel)


def kernel(positive_triplets, negative_triplets, entity_emb, relation_emb):
    trip = jnp.concatenate([positive_triplets, negative_triplets], axis=0)
    trip = trip.astype(jnp.int32)
    hidx = trip[:, 0]
    ridx = trip[:, 1]
    tidx = trip[:, 2]
    out = _trans_e(hidx, ridx, tidx, entity_emb, relation_emb)
    return out[:BATCH], out[BATCH:]


# speedup vs baseline: 1.9423x; 1.0546x over previous
import functools

import jax
import jax.numpy as jnp
from jax import lax
from jax.experimental import pallas as pl
from jax.experimental.pallas import tpu as pltpu
from jax.experimental.pallas import tpu_sc as plsc

ENTITY_SIZE = 1000000
EMB = 64
BATCH = 16384
TOTAL = 2 * BATCH

NC = 2
NS = 16
NW = NC * NS
L = 16

PER_W = TOTAL // NW
CH = 256
GROUPS = CH // L


def _rsqrt_nr(x):
    i = plsc.bitcast(x, jnp.int32)
    i = jnp.int32(0x5F3759DF) - lax.shift_right_logical(i, 1)
    y = plsc.bitcast(i, jnp.float32)
    half = x * jnp.float32(0.5)
    for _ in range(3):
        y = y * (jnp.float32(1.5) - half * y * y)
    return y


def _body(hidx_hbm, ridx_hbm, tidx_hbm, ent_hbm, rel_hbm, out_hbm,
          hidx_v, ridx_v, tidx_v, hrows, rrows, trows, out_v,
          semh, semr, semt):
    wid = lax.axis_index("s") * NC + lax.axis_index("c")
    lane = lax.broadcasted_iota(jnp.int32, (L,), 0)
    zeros = jnp.zeros((L,), jnp.float32)
    ones = jnp.full((L,), 1.0, jnp.float32)
    last = jnp.full((L,), ENTITY_SIZE - 1, jnp.int32)

    for chunk in range(PER_W // CH):
        base = wid * PER_W + chunk * CH
        pltpu.sync_copy(hidx_hbm.at[pl.ds(base, CH)], hidx_v)
        pltpu.sync_copy(ridx_hbm.at[pl.ds(base, CH)], ridx_v)
        pltpu.sync_copy(tidx_hbm.at[pl.ds(base, CH)], tidx_v)
        ch = pltpu.async_copy(ent_hbm.at[hidx_v], hrows, semh)
        cr = pltpu.async_copy(rel_hbm.at[ridx_v], rrows, semr)
        ct = pltpu.async_copy(ent_hbm.at[tidx_v], trows, semt)
        ch.wait()
        cr.wait()
        ct.wait()

        def group(g, carry):
            rows = g * L + lane
            hidx = hidx_v[pl.ds(g * L, L)]
            tidx = tidx_v[pl.ds(g * L, L)]

            def sums(c, accs):
                shh, srr, stt, shr, srt, sht = accs
                for u in range(4):
                    cols = jnp.full((L,), 0, jnp.int32) + (c * 4 + u)
                    gh = plsc.load_gather(hrows, [rows, cols])
                    gr = plsc.load_gather(rrows, [rows, cols])
                    gt = plsc.load_gather(trows, [rows, cols])
                    shh = shh + gh * gh
                    srr = srr + gr * gr
                    stt = stt + gt * gt
                    shr = shr + gh * gr
                    srt = srt + gr * gt
                    sht = sht + gh * gt
                return shh, srr, stt, shr, srt, sht

            shh, srr, stt, shr, srt, sht = lax.fori_loop(
                0, EMB // 4, sums, (zeros,) * 6
            )
            sh = jnp.where(hidx == last, ones, _rsqrt_nr(shh))
            st = jnp.where(tidx == last, ones, _rsqrt_nr(stt))
            acc = (
                sh * sh * shh + srr + st * st * stt
                + 2.0 * (sh * shr - st * srt - sh * st * sht)
            )
            dist = jnp.where(acc > 0, acc * _rsqrt_nr(acc), zeros)
            out_v[pl.ds(g * L, L)] = dist
            return carry

        lax.fori_loop(0, GROUPS, group, 0)
        pltpu.sync_copy(out_v, out_hbm.at[pl.ds(base, CH)])


@functools.partial(jax.jit, donate_argnums=())
def _trans_e(hidx, ridx, tidx, ent, rel):
    run = functools.partial(
        pl.kernel,
        out_type=jax.ShapeDtypeStruct((TOTAL,), jnp.float32),
        mesh=plsc.VectorSubcoreMesh(core_axis_name="c", subcore_axis_name="s"),
        scratch_types=[
            pltpu.VMEM((CH,), jnp.int32),
            pltpu.VMEM((CH,), jnp.int32),
            pltpu.VMEM((CH,), jnp.int32),
            pltpu.VMEM((CH, EMB), jnp.float32),
            pltpu.VMEM((CH, EMB), jnp.float32),
            pltpu.VMEM((CH, EMB), jnp.float32),
            pltpu.VMEM((CH,), jnp.float32),
            pltpu.SemaphoreType.DMA,
            pltpu.SemaphoreType.DMA,
            pltpu.SemaphoreType.DMA,
        ],
        compiler_params=pltpu.CompilerParams(
            needs_layout_passes=False, use_tc_tiling_on_sc=False
        ),
    )(_body)
    return run(hidx, ridx, tidx, ent, rel)


def kernel(positive_triplets, negative_triplets, entity_emb, relation_emb):
    trip = jnp.concatenate([positive_triplets, negative_triplets], axis=0)
    trip = trip.astype(jnp.int32)
    hidx = trip[:, 0]
    ridx = trip[:, 1]
    tidx = trip[:, 2]
    out = _trans_e(hidx, ridx, tidx, entity_emb, relation_emb)
    return out[:BATCH], out[BATCH:]
